# baseline (speedup 1.0000x reference)
import functools

import jax
import jax.numpy as jnp
from jax.experimental import pallas as pl
from jax.experimental.pallas import tpu as pltpu


def _body(g_ref, ft_ref, w0_ref, w1_ref, w2_ref, b0_ref, b1_ref, b2_ref,
          out_ref, a16_ref, r_ref, s_ref, u_ref, acc_ref, za_ref, zb_ref,
          *, bi, ni):
    l = pl.program_id(0)
    i = pl.program_id(1)
    isl = pl.ds(i * bi, bi)

    @pl.when(l == 0)
    def _():
        a = g_ref[...]
        row = jax.lax.broadcasted_iota(jnp.int32, a.shape, 0) + i * bi
        col = jax.lax.broadcasted_iota(jnp.int32, a.shape, 1)
        a = jnp.where(row == col, 1.0, a)
        a16_ref[isl, :] = a.astype(jnp.bfloat16)
        r_ref[isl, :] = jnp.sum(a, axis=1, keepdims=True)

    @pl.when(l == 1)
    def _():
        p = jax.lax.rsqrt(r_ref[isl, :])
        part = jax.lax.dot_general(
            a16_ref[isl, :].astype(jnp.float32), p,
            (((0,), (0,)), ((), ())),
            preferred_element_type=jnp.float32,
        )

        @pl.when(i == 0)
        def _():
            s_ref[...] = part

        @pl.when(i != 0)
        def _():
            s_ref[...] = s_ref[...] + part

        @pl.when(i == ni - 1)
        def _():
            u_ref[...] = jax.lax.rsqrt(jnp.sqrt(r_ref[...]) * s_ref[...])

    def layer(zsrc, w_ref, b_ref, writeback):
        fout = w_ref.shape[1]
        y = jnp.dot(zsrc.astype(jnp.float32), w_ref[...],
                    preferred_element_type=jnp.float32)
        y = (y * u_ref[isl, :]).astype(jnp.bfloat16)
        part = jax.lax.dot_general(
            a16_ref[isl, :], y, (((0,), (0,)), ((), ())),
            preferred_element_type=jnp.float32,
        )

        @pl.when(i == 0)
        def _():
            acc_ref[:, :fout] = part

        @pl.when(i != 0)
        def _():
            acc_ref[:, :fout] = acc_ref[:, :fout] + part

        @pl.when(i == ni - 1)
        def _():
            writeback(jnp.maximum(
                acc_ref[:, :fout] * u_ref[...] + b_ref[...], 0.0))

    @pl.when(l == 2)
    def _():
        layer(ft_ref[...], w0_ref, b0_ref,
              lambda v: za_ref.__setitem__((Ellipsis,), v.astype(jnp.bfloat16)))

    @pl.when(l == 3)
    def _():
        layer(za_ref[isl, :], w1_ref, b1_ref,
              lambda v: zb_ref.__setitem__((Ellipsis,), v.astype(jnp.bfloat16)))

    @pl.when(l == 4)
    def _():
        layer(zb_ref[isl, :], w2_ref, b2_ref,
              lambda v: out_ref.__setitem__((Ellipsis,), v))


def kernel(features, graph, W0, b0, W1, b1, W2, b2):
    n = graph.shape[0]
    bi = 128
    ni = n // bi
    d0, h = W0.shape
    latent = W2.shape[1]

    full = lambda shape: pl.BlockSpec(shape, lambda l, i: (0, 0))
    out = pl.pallas_call(
        functools.partial(_body, bi=bi, ni=ni),
        grid=(5, ni),
        in_specs=[
            pl.BlockSpec((bi, n), lambda l, i: (jnp.where(l == 0, i, 0), 0)),
            pl.BlockSpec((bi, d0), lambda l, i: (jnp.where(l == 2, i, 0), 0)),
            full((d0, h)),
            full((h, h)),
            full((h, latent)),
            full((1, h)),
            full((1, h)),
            full((1, latent)),
        ],
        out_specs=full((n, latent)),
        out_shape=jax.ShapeDtypeStruct((n, latent), jnp.float32),
        scratch_shapes=[
            pltpu.VMEM((n, n), jnp.bfloat16),
            pltpu.VMEM((n, 1), jnp.float32),
            pltpu.VMEM((n, 1), jnp.float32),
            pltpu.VMEM((n, 1), jnp.float32),
            pltpu.VMEM((n, h), jnp.float32),
            pltpu.VMEM((n, h), jnp.bfloat16),
            pltpu.VMEM((n, h), jnp.bfloat16),
        ],
        compiler_params=pltpu.CompilerParams(
            dimension_semantics=("arbitrary", "arbitrary")
        ),
    )(graph, features, W0, W1, W2,
      b0.reshape(1, h), b1.reshape(1, h), b2.reshape(1, latent))
    return out
